# Initial kernel scaffold; baseline (speedup 1.0000x reference)
import functools

import jax
import jax.numpy as jnp
from jax import lax
from jax.experimental import pallas as pl
from jax.experimental.pallas import tpu as pltpu
from jax.experimental.pallas import tpu_sc as plsc

_N = 10000
_E = 320000
_D = 128

_NC = 2
_NS = 16
_NW = _NC * _NS
_EPW = _E // _NW
_CHUNK = 80
_NCHUNK = _EPW // _CHUNK
_ZROWS = 125
_RPS = _N // _NS


def _sc_aggregate(x, src, dst):
    mesh = plsc.VectorSubcoreMesh(core_axis_name="c", subcore_axis_name="s")

    @functools.partial(
        pl.kernel,
        mesh=mesh,
        out_type=jax.ShapeDtypeStruct((2 * _N, _D), jnp.float32),
        scratch_types=[
            pltpu.VMEM((_CHUNK,), jnp.int32),
            pltpu.VMEM((_CHUNK,), jnp.int32),
            pltpu.VMEM((_CHUNK, _D), jnp.float32),
            pltpu.VMEM((_ZROWS, _D), jnp.float32),
            pltpu.VMEM_SHARED((_N, _D), jnp.float32),
            pltpu.SemaphoreType.DMA,
        ],
    )
    def agg(x_hbm, src_hbm, dst_hbm, out_hbm, src_v, dst_v, rows_v, z_v, acc_s, sem):
        c = lax.axis_index("c")
        s = lax.axis_index("s")
        wid = c * _NS + s

        def zstore(i, _):
            z_v[i // (_D // 16), pl.ds((i % (_D // 16)) * 16, 16)] = jnp.zeros(
                (16,), jnp.float32)
            return _

        lax.fori_loop(0, _ZROWS * (_D // 16), zstore, None)

        def zcopy(j, _):
            pltpu.sync_copy(z_v, acc_s.at[pl.ds(s * _RPS + j * _ZROWS, _ZROWS)])
            return _

        lax.fori_loop(0, _RPS // _ZROWS, zcopy, None)
        plsc.subcore_barrier()

        def chunk(j, _):
            base = wid * _EPW + j * _CHUNK
            pltpu.sync_copy(src_hbm.at[pl.ds(base, _CHUNK)], src_v)
            pltpu.sync_copy(dst_hbm.at[pl.ds(base, _CHUNK)], dst_v)
            pltpu.async_copy(x_hbm.at[src_v], rows_v, sem).wait()
            pltpu.sync_copy(rows_v, acc_s.at[dst_v], add=True)
            return _

        lax.fori_loop(0, _NCHUNK, chunk, None)
        plsc.subcore_barrier()

        pltpu.sync_copy(
            acc_s.at[pl.ds(s * _RPS, _RPS)],
            out_hbm.at[pl.ds(c * _N + s * _RPS, _RPS)],
        )

    return agg(x, src, dst)


def _tc_combine(partials, W, b2):
    bn = 1000
    grid = (_N // bn,)

    def body(p0_ref, p1_ref, w_ref, b_ref, o_ref):
        a = p0_ref[...] + p1_ref[...]
        h = lax.dot_general(a, w_ref[...], (((1,), (1,)), ((), ())),
                            preferred_element_type=jnp.float32)
        o_ref[...] = h + b_ref[...]

    return pl.pallas_call(
        body,
        grid=grid,
        in_specs=[
            pl.BlockSpec((bn, _D), lambda i: (i, 0)),
            pl.BlockSpec((bn, _D), lambda i: (i + _N // bn, 0)),
            pl.BlockSpec((_D, _D), lambda i: (0, 0)),
            pl.BlockSpec((1, _D), lambda i: (0, 0)),
        ],
        out_specs=pl.BlockSpec((bn, _D), lambda i: (i, 0)),
        out_shape=jax.ShapeDtypeStruct((_N, _D), jnp.float32),
    )(partials, partials, W, b2)


@jax.jit
def kernel(x, edge_index, W, b):
    src = edge_index[0]
    dst = edge_index[1]
    partials = _sc_aggregate(x, src, dst)
    out = _tc_combine(partials, W, b.reshape(1, _D))
    return (out,)


# speedup vs baseline: 5.5140x; 5.5140x over previous
import functools

import jax
import jax.numpy as jnp
from jax import lax
from jax.experimental import pallas as pl
from jax.experimental.pallas import tpu as pltpu
from jax.experimental.pallas import tpu_sc as plsc

_N = 10000
_E = 320000
_D = 128

_NC = 2
_NS = 16
_NW = _NC * _NS
_EPW = _E // _NW
_CHUNK = 80
_NCHUNK = _EPW // _CHUNK
_NPAD = 10240
_ZROWS = 128
_RPS = _NPAD // _NS


def _sc_aggregate(x, src, dst):
    mesh = plsc.VectorSubcoreMesh(core_axis_name="c", subcore_axis_name="s")

    @functools.partial(
        pl.kernel,
        mesh=mesh,
        out_type=jax.ShapeDtypeStruct((2, _NPAD, _D), jnp.float32),
        scratch_types=[
            pltpu.VMEM((_CHUNK,), jnp.int32),
            pltpu.VMEM((_CHUNK,), jnp.int32),
            pltpu.VMEM((_CHUNK, _D), jnp.float32),
            pltpu.VMEM((_ZROWS, _D), jnp.float32),
            pltpu.VMEM_SHARED((_NPAD, _D), jnp.float32),
            pltpu.SemaphoreType.DMA,
        ],
    )
    def agg(x_hbm, src_hbm, dst_hbm, out_hbm, src_v, dst_v, rows_v, z_v, acc_s, sem):
        c = lax.axis_index("c")
        s = lax.axis_index("s")
        wid = c * _NS + s

        def zstore(i, _):
            z_v[i // (_D // 16), pl.ds((i % (_D // 16)) * 16, 16)] = jnp.zeros(
                (16,), jnp.float32)
            return _

        lax.fori_loop(0, _ZROWS * (_D // 16), zstore, None)

        def zcopy(j, _):
            pltpu.sync_copy(z_v, acc_s.at[pl.ds(s * _RPS + j * _ZROWS, _ZROWS)])
            return _

        lax.fori_loop(0, _RPS // _ZROWS, zcopy, None)
        plsc.subcore_barrier()

        def chunk(j, _):
            base = wid * _EPW + j * _CHUNK
            pltpu.sync_copy(src_hbm.at[pl.ds(base, _CHUNK)], src_v)
            pltpu.sync_copy(dst_hbm.at[pl.ds(base, _CHUNK)], dst_v)
            pltpu.async_copy(x_hbm.at[src_v], rows_v, sem).wait()
            pltpu.sync_copy(rows_v, acc_s.at[dst_v], add=True)
            return _

        lax.fori_loop(0, _NCHUNK, chunk, None)
        plsc.subcore_barrier()

        pltpu.sync_copy(
            acc_s.at[pl.ds(s * _RPS, _RPS)],
            out_hbm.at[c, pl.ds(s * _RPS, _RPS)],
        )

    return agg(x, src, dst)


def _tc_combine(partials, W, b2):
    bn = 1000
    grid = (_N // bn,)

    def body(p0_ref, p1_ref, w_ref, b_ref, o_ref):
        a = p0_ref[0] + p1_ref[0]
        h = lax.dot_general(a, w_ref[...], (((1,), (1,)), ((), ())),
                            preferred_element_type=jnp.float32)
        o_ref[...] = h + b_ref[...]

    return pl.pallas_call(
        body,
        grid=grid,
        in_specs=[
            pl.BlockSpec((1, bn, _D), lambda i: (0, i, 0)),
            pl.BlockSpec((1, bn, _D), lambda i: (1, i, 0)),
            pl.BlockSpec((_D, _D), lambda i: (0, 0)),
            pl.BlockSpec((1, _D), lambda i: (0, 0)),
        ],
        out_specs=pl.BlockSpec((bn, _D), lambda i: (i, 0)),
        out_shape=jax.ShapeDtypeStruct((_N, _D), jnp.float32),
    )(partials, partials, W, b2)


@jax.jit
def kernel(x, edge_index, W, b):
    src = edge_index[0]
    dst = edge_index[1]
    partials = _sc_aggregate(x, src, dst)
    out = _tc_combine(partials, W, b.reshape(1, _D))
    return (out,)
